# baseline (speedup 1.0000x reference)
import jax
import jax.numpy as jnp
from jax import lax
from jax.experimental import pallas as pl
from jax.experimental.pallas import tpu as pltpu
from jax.experimental.pallas import tpu_sc as plsc

N_NODES = 10000
N_PAD = 10240
N_EDGES = 320000
D_FEAT = 128
D_AUG = 144
D_HID = 256
NC = 2
NS = 16
CH = 80
NCHUNK = N_EDGES // CH
RPT = N_PAD // NS
RB = 2048

_f32 = jnp.float32


HC = CH // 2


def _edge_pass(tbl_hbm, eidx_hbm, acc_sh,
               idx0, idx1, rows0, rows1, g0, g1, q0, q1, s0, s1, k0, n):
    idx = (idx0, idx1)
    rows = (rows0, rows1)
    gsem = (g0, g1)
    qsem = (q0, q1)

    def prefetch(i, b):
        ro = 2 * (k0 + jnp.minimum(i, n - 1))
        pltpu.async_copy(eidx_hbm.at[:, pl.ds(ro, 2)], idx[b], qsem[b])

    def wait_idx(b):
        pltpu.make_async_copy(eidx_hbm.at[:, pl.ds(2 * k0, 2)], idx[b],
                              qsem[b]).wait()

    def gather(b):
        pltpu.async_copy(tbl_hbm.at[idx[b].at[0, 0]],
                         rows[b].at[pl.ds(0, HC)], gsem[b])
        pltpu.async_copy(tbl_hbm.at[idx[b].at[0, 1]],
                         rows[b].at[pl.ds(HC, HC)], gsem[b])

    def wait_gather(b):
        pltpu.make_async_copy(tbl_hbm.at[idx[b].at[0, 0]],
                              rows[b].at[pl.ds(0, HC)], gsem[b]).wait()
        pltpu.make_async_copy(tbl_hbm.at[idx[b].at[0, 1]],
                              rows[b].at[pl.ds(HC, HC)], gsem[b]).wait()

    def scatter(b):
        d0 = pltpu.async_copy(rows[b].at[pl.ds(0, HC)],
                              acc_sh.at[idx[b].at[1, 0]], s0, add=True)
        d1 = pltpu.async_copy(rows[b].at[pl.ds(HC, HC)],
                              acc_sh.at[idx[b].at[1, 1]], s1, add=True)
        d0.wait()
        d1.wait()

    def block(i, b):
        wait_idx(1 - b)
        gather(1 - b)
        wait_gather(b)
        scatter(b)
        prefetch(i + 2, b)

    prefetch(0, 0)
    wait_idx(0)
    gather(0)
    prefetch(1, 1)

    pairs = (n - 1) // 2

    @pl.loop(0, pairs)
    def _(g):
        i0 = 2 * g
        block(i0, 0)
        block(i0 + 1, 1)

    for i in range(2 * pairs, n - 1):
        block(i, i % 2)

    bl = (n - 1) % 2
    wait_gather(bl)
    scatter(bl)
    wait_idx(1 - bl)


def _agg1_body(xa_hbm, eidx_hbm, zer_hbm, outa, outb,
               acc, idx0, idx1, rows0, rows1, g0, g1, q0, q1, s0, s1):
    c = lax.axis_index("c")
    s = lax.axis_index("s")
    r0 = s * RPT
    pltpu.sync_copy(zer_hbm.at[pl.ds(r0, RPT)], acc.at[pl.ds(r0, RPT)])
    plsc.subcore_barrier()

    cpw = NCHUNK // (NC * NS)
    _edge_pass(xa_hbm, eidx_hbm, acc, idx0, idx1, rows0, rows1,
               g0, g1, q0, q1, s0, s1, (c * NS + s) * cpw, cpw)
    plsc.subcore_barrier()

    @pl.when(c == 0)
    def _():
        pltpu.sync_copy(acc.at[pl.ds(r0, RPT)], outa.at[pl.ds(r0, RPT)])

    @pl.when(c == 1)
    def _():
        pltpu.sync_copy(acc.at[pl.ds(r0, RPT)], outb.at[pl.ds(r0, RPT)])


def _agg2_body(h1a_hbm, h1b_hbm, eidx_hbm, zer_hbm, outa, outb,
               acc, idx0, idx1, rows0, rows1, g0, g1, q0, q1, s0, s1):
    c = lax.axis_index("c")
    s = lax.axis_index("s")
    r0 = s * RPT
    pltpu.sync_copy(zer_hbm.at[pl.ds(r0, RPT)], acc.at[pl.ds(r0, RPT)])
    plsc.subcore_barrier()

    cpw = NCHUNK // NS
    k0 = s * cpw

    @pl.when(c == 0)
    def _():
        _edge_pass(h1a_hbm, eidx_hbm, acc, idx0, idx1, rows0, rows1,
                   g0, g1, q0, q1, s0, s1, k0, cpw)

    @pl.when(c == 1)
    def _():
        _edge_pass(h1b_hbm, eidx_hbm, acc, idx0, idx1, rows0, rows1,
                   g0, g1, q0, q1, s0, s1, k0, cpw)

    plsc.subcore_barrier()

    @pl.when(c == 0)
    def _():
        pltpu.sync_copy(acc.at[pl.ds(r0, RPT)], outa.at[pl.ds(r0, RPT)])

    @pl.when(c == 1)
    def _():
        pltpu.sync_copy(acc.at[pl.ds(r0, RPT)], outb.at[pl.ds(r0, RPT)])


def _dot(a, b):
    return jnp.dot(a, b, preferred_element_type=_f32,
                   precision=lax.Precision.HIGHEST)


def _xr_body(x_ref, w_ref, b_ref, out_ref):
    out_ref[...] = _dot(x_ref[...], w_ref[...]) + b_ref[...]


def _lin2_body(xa_ref, xb_ref, wa_ref, wb_ref, b_ref, out_ref):
    out_ref[...] = (_dot(xa_ref[...], wa_ref[...])
                    + _dot(xb_ref[...], wb_ref[...]) + b_ref[...])


def _l1_post_body(pa, pb, xr, wl, ha, hb, rinv):
    t = pa[...] + pb[...]
    cnt = t[:, D_FEAT:D_FEAT + 1]
    r = 1.0 / jnp.maximum(cnt, 1.0)
    agg = t[:, :D_FEAT] * r
    h = jnp.maximum(_dot(agg, wl[...]) + xr[...], 0.0)
    ha[...] = h[:, :D_FEAT]
    hb[...] = h[:, D_FEAT:]
    rinv[...] = r


def _l2_post_body(s2a, s2b, yr, rinv, wl0, wl1, wfc, bfc, out):
    r = rinv[...]
    z = (_dot(s2a[...] * r, wl0[...]) + _dot(s2b[...] * r, wl1[...])
         + yr[...])
    h2 = jnp.maximum(z, 0.0)
    out[...] = _dot(h2, wfc[...]) + bfc[...]


def _row_spec(d):
    return pl.BlockSpec((RB, d), lambda i: (i, 0))


def _full_spec(r, d):
    return pl.BlockSpec((r, d), lambda i: (0, 0))


def kernel(x, edge_index, W1l, b1l, W1r, W2l, b2l, W2r, Wfc, bfc):
    eidx = edge_index.astype(jnp.int32).reshape(2, 2 * NCHUNK, HC)
    xp = jnp.pad(x, ((0, N_PAD - N_NODES), (0, 0)))
    xa = jnp.concatenate(
        [xp, jnp.ones((N_PAD, D_AUG - D_FEAT), _f32)], axis=1)
    z144 = jnp.zeros((N_PAD, D_AUG), _f32)
    z128 = jnp.zeros((N_PAD, D_FEAT), _f32)

    mesh = plsc.VectorSubcoreMesh(core_axis_name="c", subcore_axis_name="s")
    sc_params = pltpu.CompilerParams(use_tc_tiling_on_sc=False)

    def sc_scratch(d):
        return [pltpu.VMEM_SHARED((N_PAD, d), _f32),
                pltpu.VMEM((2, 2, HC), jnp.int32),
                pltpu.VMEM((2, 2, HC), jnp.int32),
                pltpu.VMEM((CH, d), _f32),
                pltpu.VMEM((CH, d), _f32),
                pltpu.SemaphoreType.DMA,
                pltpu.SemaphoreType.DMA,
                pltpu.SemaphoreType.DMA,
                pltpu.SemaphoreType.DMA,
                pltpu.SemaphoreType.DMA,
                pltpu.SemaphoreType.DMA]

    grid1 = (N_PAD // RB,)

    xr = pl.pallas_call(
        _xr_body,
        grid=grid1,
        in_specs=[_row_spec(D_FEAT), _full_spec(D_FEAT, D_HID),
                  _full_spec(1, D_HID)],
        out_specs=[_row_spec(D_HID)],
        out_shape=[jax.ShapeDtypeStruct((N_PAD, D_HID), _f32)],
    )(xp, W1r.T, b1l[None, :])[0]

    agg1 = pl.kernel(
        _agg1_body,
        out_type=[jax.ShapeDtypeStruct((N_PAD, D_AUG), _f32),
                  jax.ShapeDtypeStruct((N_PAD, D_AUG), _f32)],
        mesh=mesh,
        scratch_types=sc_scratch(D_AUG),
        compiler_params=sc_params,
    )
    pa, pb = agg1(xa, eidx, z144)

    ha, hb, rinv = pl.pallas_call(
        _l1_post_body,
        grid=grid1,
        in_specs=[_row_spec(D_AUG), _row_spec(D_AUG), _row_spec(D_HID),
                  _full_spec(D_FEAT, D_HID)],
        out_specs=[_row_spec(D_FEAT), _row_spec(D_FEAT),
                   pl.BlockSpec((RB, 1), lambda i: (i, 0))],
        out_shape=[jax.ShapeDtypeStruct((N_PAD, D_FEAT), _f32),
                   jax.ShapeDtypeStruct((N_PAD, D_FEAT), _f32),
                   jax.ShapeDtypeStruct((N_PAD, 1), _f32)],
    )(pa, pb, xr, W1l.T)

    w2lT = W2l.T
    w2rT = W2r.T

    yr = pl.pallas_call(
        _lin2_body,
        grid=grid1,
        in_specs=[_row_spec(D_FEAT), _row_spec(D_FEAT),
                  _full_spec(D_FEAT, D_HID), _full_spec(D_FEAT, D_HID),
                  _full_spec(1, D_HID)],
        out_specs=[_row_spec(D_HID)],
        out_shape=[jax.ShapeDtypeStruct((N_PAD, D_HID), _f32)],
    )(ha, hb, w2rT[:D_FEAT], w2rT[D_FEAT:], b2l[None, :])[0]

    agg2 = pl.kernel(
        _agg2_body,
        out_type=[jax.ShapeDtypeStruct((N_PAD, D_FEAT), _f32),
                  jax.ShapeDtypeStruct((N_PAD, D_FEAT), _f32)],
        mesh=mesh,
        scratch_types=sc_scratch(D_FEAT),
        compiler_params=sc_params,
    )
    s2a, s2b = agg2(ha, hb, eidx, z128)

    out = pl.pallas_call(
        _l2_post_body,
        grid=grid1,
        in_specs=[_row_spec(D_FEAT), _row_spec(D_FEAT), _row_spec(D_HID),
                  pl.BlockSpec((RB, 1), lambda i: (i, 0)),
                  _full_spec(D_FEAT, D_HID), _full_spec(D_FEAT, D_HID),
                  _full_spec(D_HID, 1), _full_spec(1, 1)],
        out_specs=[pl.BlockSpec((RB, 1), lambda i: (i, 0))],
        out_shape=[jax.ShapeDtypeStruct((N_PAD, 1), _f32)],
    )(s2a, s2b, yr, rinv,
      w2lT[:D_FEAT], w2lT[D_FEAT:], Wfc.T, bfc[None, :])[0]

    return out[:N_NODES]


# speedup vs baseline: 1.1365x; 1.1365x over previous
import jax
import jax.numpy as jnp
from jax import lax
from jax.experimental import pallas as pl
from jax.experimental.pallas import tpu as pltpu
from jax.experimental.pallas import tpu_sc as plsc

N_NODES = 10000
N_PAD = 10240
N_EDGES = 320000
D_FEAT = 128
D_CNT = 16
D_HID = 256
NC = 2
NS = 16
CH = 80
NCHUNK = N_EDGES // CH
RPT = N_PAD // NS
RB = 2048

_f32 = jnp.float32


def _edge_pass(tbl_hbm, eidx_hbm, acc_sh, idx, rows, gsem, qsem,
               base, n, cnt=None):
    def prefetch(i, b):
        off = base + jnp.minimum(i, n - 1) * CH
        pltpu.async_copy(eidx_hbm.at[:, pl.ds(off, CH)], idx[b], qsem[b])

    def wait_idx(b):
        pltpu.make_async_copy(eidx_hbm.at[:, pl.ds(base, CH)], idx[b],
                              qsem[b]).wait()

    def gather(b):
        pltpu.async_copy(tbl_hbm.at[idx[b].at[0]], rows[b], gsem[b])

    def finish(b):
        pltpu.make_async_copy(tbl_hbm.at[idx[b].at[0]], rows[b],
                              gsem[b]).wait()
        if cnt is not None:
            cacc, ones_v, csem = cnt
            pltpu.async_copy(ones_v, cacc.at[idx[b].at[1]], csem,
                             add=True)
        pltpu.sync_copy(rows[b], acc_sh.at[idx[b].at[1]], add=True)
        if cnt is not None:
            cacc, ones_v, csem = cnt
            pltpu.make_async_copy(ones_v, cacc.at[idx[b].at[1]],
                                  csem).wait()

    def block(i, b):
        wait_idx(1 - b)
        gather(1 - b)
        finish(b)
        prefetch(i + 2, b)

    prefetch(0, 0)
    wait_idx(0)
    gather(0)
    prefetch(1, 1)

    pairs = (n - 1) // 2

    @pl.loop(0, pairs)
    def _(g):
        i0 = 2 * g
        block(i0, 0)
        block(i0 + 1, 1)

    for i in range(2 * pairs, n - 1):
        block(i, i % 2)

    bl = (n - 1) % 2
    finish(bl)
    wait_idx(1 - bl)


def _agg1_body(xp_hbm, eidx_hbm, z128_hbm, z16_hbm, ones_hbm,
               outa, outb, outca, outcb,
               acc, cacc, idx0, idx1, rows0, rows1, ones_v,
               g0, g1, q0, q1, cs):
    c = lax.axis_index("c")
    s = lax.axis_index("s")
    r0 = s * RPT
    pltpu.sync_copy(z128_hbm.at[pl.ds(r0, RPT)], acc.at[pl.ds(r0, RPT)])
    pltpu.sync_copy(z16_hbm.at[pl.ds(r0, RPT)], cacc.at[pl.ds(r0, RPT)])
    pltpu.sync_copy(ones_hbm, ones_v)
    plsc.subcore_barrier()

    cpw = NCHUNK // (NC * NS)
    _edge_pass(xp_hbm, eidx_hbm, acc, (idx0, idx1), (rows0, rows1),
               (g0, g1), (q0, q1), (c * NS + s) * cpw * CH, cpw,
               cnt=(cacc, ones_v, cs))
    plsc.subcore_barrier()

    @pl.when(c == 0)
    def _():
        pltpu.sync_copy(acc.at[pl.ds(r0, RPT)], outa.at[pl.ds(r0, RPT)])
        pltpu.sync_copy(cacc.at[pl.ds(r0, RPT)], outca.at[pl.ds(r0, RPT)])

    @pl.when(c == 1)
    def _():
        pltpu.sync_copy(acc.at[pl.ds(r0, RPT)], outb.at[pl.ds(r0, RPT)])
        pltpu.sync_copy(cacc.at[pl.ds(r0, RPT)], outcb.at[pl.ds(r0, RPT)])


def _agg2_body(h1a_hbm, h1b_hbm, eidx_hbm, zer_hbm, outa, outb,
               acc, idx0, idx1, rows0, rows1, g0, g1, q0, q1):
    c = lax.axis_index("c")
    s = lax.axis_index("s")
    r0 = s * RPT
    pltpu.sync_copy(zer_hbm.at[pl.ds(r0, RPT)], acc.at[pl.ds(r0, RPT)])
    plsc.subcore_barrier()

    cpw = NCHUNK // NS
    base = s * cpw * CH

    @pl.when(c == 0)
    def _():
        _edge_pass(h1a_hbm, eidx_hbm, acc, (idx0, idx1), (rows0, rows1),
                   (g0, g1), (q0, q1), base, cpw)

    @pl.when(c == 1)
    def _():
        _edge_pass(h1b_hbm, eidx_hbm, acc, (idx0, idx1), (rows0, rows1),
                   (g0, g1), (q0, q1), base, cpw)

    plsc.subcore_barrier()

    @pl.when(c == 0)
    def _():
        pltpu.sync_copy(acc.at[pl.ds(r0, RPT)], outa.at[pl.ds(r0, RPT)])

    @pl.when(c == 1)
    def _():
        pltpu.sync_copy(acc.at[pl.ds(r0, RPT)], outb.at[pl.ds(r0, RPT)])


def _dot(a, b):
    return jnp.dot(a, b, preferred_element_type=_f32)


def _xr_body(x_ref, w_ref, b_ref, out_ref):
    out_ref[...] = _dot(x_ref[...], w_ref[...]) + b_ref[...]


def _lin2_body(xa_ref, xb_ref, wa_ref, wb_ref, b_ref, out_ref):
    out_ref[...] = (_dot(xa_ref[...], wa_ref[...])
                    + _dot(xb_ref[...], wb_ref[...]) + b_ref[...])


def _l1_post_body(pa, pb, ca, cb, xr, wl, ha, hb, rinv):
    t = pa[...] + pb[...]
    cnt = (ca[...] + cb[...])[:, 0:1]
    r = 1.0 / jnp.maximum(cnt, 1.0)
    agg = t * r
    h = jnp.maximum(_dot(agg, wl[...]) + xr[...], 0.0)
    ha[...] = h[:, :D_FEAT]
    hb[...] = h[:, D_FEAT:]
    rinv[...] = r


def _l2_post_body(s2a, s2b, yr, rinv, wl0, wl1, wfc, bfc, out):
    r = rinv[...]
    z = (_dot(s2a[...] * r, wl0[...]) + _dot(s2b[...] * r, wl1[...])
         + yr[...])
    h2 = jnp.maximum(z, 0.0)
    out[...] = _dot(h2, wfc[...]) + bfc[...]


def _row_spec(d):
    return pl.BlockSpec((RB, d), lambda i: (i, 0))


def _full_spec(r, d):
    return pl.BlockSpec((r, d), lambda i: (0, 0))


def kernel(x, edge_index, W1l, b1l, W1r, W2l, b2l, W2r, Wfc, bfc):
    eidx = edge_index.astype(jnp.int32)
    xp = jnp.pad(x, ((0, N_PAD - N_NODES), (0, 0)))
    z128 = jnp.zeros((N_PAD, D_FEAT), _f32)
    z16 = jnp.zeros((N_PAD, D_CNT), _f32)
    ones16 = jnp.ones((CH, D_CNT), _f32)

    mesh = plsc.VectorSubcoreMesh(core_axis_name="c", subcore_axis_name="s")
    sc_params = pltpu.CompilerParams(use_tc_tiling_on_sc=False)

    grid1 = (N_PAD // RB,)

    xr = pl.pallas_call(
        _xr_body,
        grid=grid1,
        in_specs=[_row_spec(D_FEAT), _full_spec(D_FEAT, D_HID),
                  _full_spec(1, D_HID)],
        out_specs=[_row_spec(D_HID)],
        out_shape=[jax.ShapeDtypeStruct((N_PAD, D_HID), _f32)],
    )(xp, W1r.T, b1l[None, :])[0]

    agg1 = pl.kernel(
        _agg1_body,
        out_type=[jax.ShapeDtypeStruct((N_PAD, D_FEAT), _f32),
                  jax.ShapeDtypeStruct((N_PAD, D_FEAT), _f32),
                  jax.ShapeDtypeStruct((N_PAD, D_CNT), _f32),
                  jax.ShapeDtypeStruct((N_PAD, D_CNT), _f32)],
        mesh=mesh,
        scratch_types=[pltpu.VMEM_SHARED((N_PAD, D_FEAT), _f32),
                       pltpu.VMEM_SHARED((N_PAD, D_CNT), _f32),
                       pltpu.VMEM((2, CH), jnp.int32),
                       pltpu.VMEM((2, CH), jnp.int32),
                       pltpu.VMEM((CH, D_FEAT), _f32),
                       pltpu.VMEM((CH, D_FEAT), _f32),
                       pltpu.VMEM((CH, D_CNT), _f32),
                       pltpu.SemaphoreType.DMA,
                       pltpu.SemaphoreType.DMA,
                       pltpu.SemaphoreType.DMA,
                       pltpu.SemaphoreType.DMA,
                       pltpu.SemaphoreType.DMA],
        compiler_params=sc_params,
    )
    pa, pb, ca, cb = agg1(xp, eidx, z128, z16, ones16)

    ha, hb, rinv = pl.pallas_call(
        _l1_post_body,
        grid=grid1,
        in_specs=[_row_spec(D_FEAT), _row_spec(D_FEAT),
                  _row_spec(D_CNT), _row_spec(D_CNT), _row_spec(D_HID),
                  _full_spec(D_FEAT, D_HID)],
        out_specs=[_row_spec(D_FEAT), _row_spec(D_FEAT),
                   pl.BlockSpec((RB, 1), lambda i: (i, 0))],
        out_shape=[jax.ShapeDtypeStruct((N_PAD, D_FEAT), _f32),
                   jax.ShapeDtypeStruct((N_PAD, D_FEAT), _f32),
                   jax.ShapeDtypeStruct((N_PAD, 1), _f32)],
    )(pa, pb, ca, cb, xr, W1l.T)

    w2lT = W2l.T
    w2rT = W2r.T

    yr = pl.pallas_call(
        _lin2_body,
        grid=grid1,
        in_specs=[_row_spec(D_FEAT), _row_spec(D_FEAT),
                  _full_spec(D_FEAT, D_HID), _full_spec(D_FEAT, D_HID),
                  _full_spec(1, D_HID)],
        out_specs=[_row_spec(D_HID)],
        out_shape=[jax.ShapeDtypeStruct((N_PAD, D_HID), _f32)],
    )(ha, hb, w2rT[:D_FEAT], w2rT[D_FEAT:], b2l[None, :])[0]

    agg2 = pl.kernel(
        _agg2_body,
        out_type=[jax.ShapeDtypeStruct((N_PAD, D_FEAT), _f32),
                  jax.ShapeDtypeStruct((N_PAD, D_FEAT), _f32)],
        mesh=mesh,
        scratch_types=[pltpu.VMEM_SHARED((N_PAD, D_FEAT), _f32),
                       pltpu.VMEM((2, CH), jnp.int32),
                       pltpu.VMEM((2, CH), jnp.int32),
                       pltpu.VMEM((CH, D_FEAT), _f32),
                       pltpu.VMEM((CH, D_FEAT), _f32),
                       pltpu.SemaphoreType.DMA,
                       pltpu.SemaphoreType.DMA,
                       pltpu.SemaphoreType.DMA,
                       pltpu.SemaphoreType.DMA],
        compiler_params=sc_params,
    )
    s2a, s2b = agg2(ha, hb, eidx, z128)

    out = pl.pallas_call(
        _l2_post_body,
        grid=grid1,
        in_specs=[_row_spec(D_FEAT), _row_spec(D_FEAT), _row_spec(D_HID),
                  pl.BlockSpec((RB, 1), lambda i: (i, 0)),
                  _full_spec(D_FEAT, D_HID), _full_spec(D_FEAT, D_HID),
                  _full_spec(D_HID, 1), _full_spec(1, 1)],
        out_specs=[pl.BlockSpec((RB, 1), lambda i: (i, 0))],
        out_shape=[jax.ShapeDtypeStruct((N_PAD, 1), _f32)],
    )(s2a, s2b, yr, rinv,
      w2lT[:D_FEAT], w2lT[D_FEAT:], Wfc.T, bfc[None, :])[0]

    return out[:N_NODES]


# speedup vs baseline: 1.3488x; 1.1868x over previous
import jax
import jax.numpy as jnp
from jax import lax
from jax.experimental import pallas as pl
from jax.experimental.pallas import tpu as pltpu
from jax.experimental.pallas import tpu_sc as plsc

N_NODES = 10000
N_PAD = 10240
N_EDGES = 320000
D_FEAT = 128
D_CNT = 16
D_HID = 256
NC = 2
NS = 16
CH = 80
NCHUNK = N_EDGES // CH
RPT = N_PAD // NS
RB = 2048

_f32 = jnp.float32


def _edge_pass(tbl_hbm, eidx_hbm, acc_sh, idx, rows, gsem, qsem,
               base, n, cnt=None):
    def prefetch(i, b):
        off = base + jnp.minimum(i, n - 1) * CH
        pltpu.async_copy(eidx_hbm.at[:, pl.ds(off, CH)], idx[b], qsem[b])

    def wait_idx(b):
        pltpu.make_async_copy(eidx_hbm.at[:, pl.ds(base, CH)], idx[b],
                              qsem[b]).wait()

    def gather(b):
        pltpu.async_copy(tbl_hbm.at[idx[b].at[0]], rows[b], gsem[b])

    def finish(b):
        pltpu.make_async_copy(tbl_hbm.at[idx[b].at[0]], rows[b],
                              gsem[b]).wait()
        if cnt is not None:
            cacc, ones_v, csem = cnt
            pltpu.async_copy(ones_v, cacc.at[idx[b].at[1]], csem,
                             add=True)
        pltpu.sync_copy(rows[b], acc_sh.at[idx[b].at[1]], add=True)
        if cnt is not None:
            cacc, ones_v, csem = cnt
            pltpu.make_async_copy(ones_v, cacc.at[idx[b].at[1]],
                                  csem).wait()

    def block(i, b):
        wait_idx(1 - b)
        gather(1 - b)
        finish(b)
        prefetch(i + 2, b)

    prefetch(0, 0)
    wait_idx(0)
    gather(0)
    prefetch(1, 1)

    pairs = (n - 1) // 2

    @pl.loop(0, pairs)
    def _(g):
        i0 = 2 * g
        block(i0, 0)
        block(i0 + 1, 1)

    for i in range(2 * pairs, n - 1):
        block(i, i % 2)

    bl = (n - 1) % 2
    finish(bl)
    wait_idx(1 - bl)


def _agg1_body(xp_hbm, eidx_hbm, z128_hbm, z16_hbm, ones_hbm,
               outa, outb, outca, outcb,
               acc, cacc, idx0, idx1, rows0, rows1, ones_v,
               g0, g1, q0, q1, cs):
    c = lax.axis_index("c")
    s = lax.axis_index("s")
    r0 = s * RPT
    pltpu.sync_copy(z128_hbm.at[pl.ds(r0, RPT)], acc.at[pl.ds(r0, RPT)])
    pltpu.sync_copy(z16_hbm.at[pl.ds(r0, RPT)], cacc.at[pl.ds(r0, RPT)])
    pltpu.sync_copy(ones_hbm, ones_v)
    plsc.subcore_barrier()

    cpw = NCHUNK // (NC * NS)
    _edge_pass(xp_hbm, eidx_hbm, acc, (idx0, idx1), (rows0, rows1),
               (g0, g1), (q0, q1), (c * NS + s) * cpw * CH, cpw,
               cnt=(cacc, ones_v, cs))
    plsc.subcore_barrier()

    @pl.when(c == 0)
    def _():
        pltpu.sync_copy(acc.at[pl.ds(r0, RPT)], outa.at[pl.ds(r0, RPT)])
        pltpu.sync_copy(cacc.at[pl.ds(r0, RPT)], outca.at[pl.ds(r0, RPT)])

    @pl.when(c == 1)
    def _():
        pltpu.sync_copy(acc.at[pl.ds(r0, RPT)], outb.at[pl.ds(r0, RPT)])
        pltpu.sync_copy(cacc.at[pl.ds(r0, RPT)], outcb.at[pl.ds(r0, RPT)])


def _agg2_body(h1bf_hbm, eidx_hbm, zer_hbm, outa, outb,
               acc, idx0, idx1, rows0, rows1, g0, g1, q0, q1):
    c = lax.axis_index("c")
    s = lax.axis_index("s")
    r0 = s * RPT
    pltpu.sync_copy(zer_hbm.at[pl.ds(r0, RPT)], acc.at[pl.ds(r0, RPT)])
    plsc.subcore_barrier()

    cpw = NCHUNK // (NC * NS)
    _edge_pass(h1bf_hbm, eidx_hbm, acc, (idx0, idx1), (rows0, rows1),
               (g0, g1), (q0, q1), (c * NS + s) * cpw * CH, cpw)
    plsc.subcore_barrier()

    @pl.when(c == 0)
    def _():
        pltpu.sync_copy(acc.at[pl.ds(r0, RPT)], outa.at[pl.ds(r0, RPT)])

    @pl.when(c == 1)
    def _():
        pltpu.sync_copy(acc.at[pl.ds(r0, RPT)], outb.at[pl.ds(r0, RPT)])


def _dot(a, b):
    return jnp.dot(a, b, preferred_element_type=_f32)


def _xr_body(x_ref, w_ref, b_ref, out_ref):
    out_ref[...] = _dot(x_ref[...], w_ref[...]) + b_ref[...]


def _lin2_body(xa_ref, xb_ref, wa_ref, wb_ref, b_ref, out_ref):
    out_ref[...] = (_dot(xa_ref[...], wa_ref[...])
                    + _dot(xb_ref[...], wb_ref[...]) + b_ref[...])


def _l1_post_body(pa, pb, ca, cb, xr, wl, ha, hb, hbf, rinv):
    t = pa[...] + pb[...]
    cnt = (ca[...] + cb[...])[:, 0:1]
    r = 1.0 / jnp.maximum(cnt, 1.0)
    agg = t * r
    h = jnp.maximum(_dot(agg, wl[...]) + xr[...], 0.0)
    ha[...] = h[:, :D_FEAT]
    hb[...] = h[:, D_FEAT:]
    hbf[...] = h.astype(jnp.bfloat16)
    rinv[...] = r


def _l2_post_body(s2a, s2b, yr, rinv, wl, wfc, bfc, out):
    ssum = s2a[...].astype(_f32) + s2b[...].astype(_f32)
    z = _dot(ssum * rinv[...], wl[...]) + yr[...]
    h2 = jnp.maximum(z, 0.0)
    out[...] = _dot(h2, wfc[...]) + bfc[...]


def _row_spec(d):
    return pl.BlockSpec((RB, d), lambda i: (i, 0))


def _full_spec(r, d):
    return pl.BlockSpec((r, d), lambda i: (0, 0))


def kernel(x, edge_index, W1l, b1l, W1r, W2l, b2l, W2r, Wfc, bfc):
    eidx = edge_index.astype(jnp.int32)
    xp = jnp.pad(x, ((0, N_PAD - N_NODES), (0, 0)))
    z128 = jnp.zeros((N_PAD, D_FEAT), _f32)
    z16 = jnp.zeros((N_PAD, D_CNT), _f32)
    ones16 = jnp.ones((CH, D_CNT), _f32)

    mesh = plsc.VectorSubcoreMesh(core_axis_name="c", subcore_axis_name="s")
    sc_params = pltpu.CompilerParams(use_tc_tiling_on_sc=False)

    grid1 = (N_PAD // RB,)

    xr = pl.pallas_call(
        _xr_body,
        grid=grid1,
        in_specs=[_row_spec(D_FEAT), _full_spec(D_FEAT, D_HID),
                  _full_spec(1, D_HID)],
        out_specs=[_row_spec(D_HID)],
        out_shape=[jax.ShapeDtypeStruct((N_PAD, D_HID), _f32)],
    )(xp, W1r.T, b1l[None, :])[0]

    agg1 = pl.kernel(
        _agg1_body,
        out_type=[jax.ShapeDtypeStruct((N_PAD, D_FEAT), _f32),
                  jax.ShapeDtypeStruct((N_PAD, D_FEAT), _f32),
                  jax.ShapeDtypeStruct((N_PAD, D_CNT), _f32),
                  jax.ShapeDtypeStruct((N_PAD, D_CNT), _f32)],
        mesh=mesh,
        scratch_types=[pltpu.VMEM_SHARED((N_PAD, D_FEAT), _f32),
                       pltpu.VMEM_SHARED((N_PAD, D_CNT), _f32),
                       pltpu.VMEM((2, CH), jnp.int32),
                       pltpu.VMEM((2, CH), jnp.int32),
                       pltpu.VMEM((CH, D_FEAT), _f32),
                       pltpu.VMEM((CH, D_FEAT), _f32),
                       pltpu.VMEM((CH, D_CNT), _f32),
                       pltpu.SemaphoreType.DMA,
                       pltpu.SemaphoreType.DMA,
                       pltpu.SemaphoreType.DMA,
                       pltpu.SemaphoreType.DMA,
                       pltpu.SemaphoreType.DMA],
        compiler_params=sc_params,
    )
    pa, pb, ca, cb = agg1(xp, eidx, z128, z16, ones16)

    ha, hb, hbf, rinv = pl.pallas_call(
        _l1_post_body,
        grid=grid1,
        in_specs=[_row_spec(D_FEAT), _row_spec(D_FEAT),
                  _row_spec(D_CNT), _row_spec(D_CNT), _row_spec(D_HID),
                  _full_spec(D_FEAT, D_HID)],
        out_specs=[_row_spec(D_FEAT), _row_spec(D_FEAT),
                   _row_spec(D_HID),
                   pl.BlockSpec((RB, 1), lambda i: (i, 0))],
        out_shape=[jax.ShapeDtypeStruct((N_PAD, D_FEAT), _f32),
                   jax.ShapeDtypeStruct((N_PAD, D_FEAT), _f32),
                   jax.ShapeDtypeStruct((N_PAD, D_HID), jnp.bfloat16),
                   jax.ShapeDtypeStruct((N_PAD, 1), _f32)],
    )(pa, pb, ca, cb, xr, W1l.T)

    w2lT = W2l.T
    w2rT = W2r.T

    yr = pl.pallas_call(
        _lin2_body,
        grid=grid1,
        in_specs=[_row_spec(D_FEAT), _row_spec(D_FEAT),
                  _full_spec(D_FEAT, D_HID), _full_spec(D_FEAT, D_HID),
                  _full_spec(1, D_HID)],
        out_specs=[_row_spec(D_HID)],
        out_shape=[jax.ShapeDtypeStruct((N_PAD, D_HID), _f32)],
    )(ha, hb, w2rT[:D_FEAT], w2rT[D_FEAT:], b2l[None, :])[0]

    zbf = jnp.zeros((N_PAD, D_HID), jnp.bfloat16)
    agg2 = pl.kernel(
        _agg2_body,
        out_type=[jax.ShapeDtypeStruct((N_PAD, D_HID), jnp.bfloat16),
                  jax.ShapeDtypeStruct((N_PAD, D_HID), jnp.bfloat16)],
        mesh=mesh,
        scratch_types=[pltpu.VMEM_SHARED((N_PAD, D_HID), jnp.bfloat16),
                       pltpu.VMEM((2, CH), jnp.int32),
                       pltpu.VMEM((2, CH), jnp.int32),
                       pltpu.VMEM((CH, D_HID), jnp.bfloat16),
                       pltpu.VMEM((CH, D_HID), jnp.bfloat16),
                       pltpu.SemaphoreType.DMA,
                       pltpu.SemaphoreType.DMA,
                       pltpu.SemaphoreType.DMA,
                       pltpu.SemaphoreType.DMA],
        compiler_params=sc_params,
    )
    s2a, s2b = agg2(hbf, eidx, zbf)

    out = pl.pallas_call(
        _l2_post_body,
        grid=grid1,
        in_specs=[_row_spec(D_HID), _row_spec(D_HID), _row_spec(D_HID),
                  pl.BlockSpec((RB, 1), lambda i: (i, 0)),
                  _full_spec(D_HID, D_HID),
                  _full_spec(D_HID, 1), _full_spec(1, 1)],
        out_specs=[pl.BlockSpec((RB, 1), lambda i: (i, 0))],
        out_shape=[jax.ShapeDtypeStruct((N_PAD, 1), _f32)],
    )(s2a, s2b, yr, rinv, w2lT, Wfc.T, bfc[None, :])[0]

    return out[:N_NODES]
